# Initial kernel scaffold; baseline (speedup 1.0000x reference)
import jax
import jax.numpy as jnp
from jax.experimental import pallas as pl


def kernel(coords, params, greedy):
    raise NotImplementedError("write your pallas kernel here")


# speedup vs baseline: 6.5817x; 6.5817x over previous
import math

import jax
import jax.numpy as jnp
from jax import lax
from jax.experimental import pallas as pl
from jax.experimental.pallas import tpu as pltpu

_B, _N, _D, _KNN, _ENC_L, _DEC_L = 32, 256, 256, 8, 2, 4
_NEG = -1e30
_HI = lax.Precision.HIGHEST


def _bdot(a, b):
    return jnp.dot(a.astype(jnp.bfloat16), b, preferred_element_type=jnp.float32)


def _encoder_body(coords_ref, win_ref, bin_ref, wqkv_ref, bqkv_ref,
                  wout_ref, bout_ref, out_ref, out_rn_ref):
    c = coords_ref[0].astype(jnp.bfloat16).astype(jnp.float32)
    w = win_ref[...].astype(jnp.float32)
    h = c[:, 0:1] * w[0:1, :] + c[:, 1:2] * w[1:2, :] + bin_ref[...]

    row = lax.broadcasted_iota(jnp.int32, (_N, _N), 0)
    col = lax.broadcasted_iota(jnp.int32, (_N, _N), 1)
    diag = row == col

    inv_sqrt_d = 1.0 / math.sqrt(_D)
    for l in range(_ENC_L):
        qkv = _bdot(h, wqkv_ref[l]) + bqkv_ref[l]
        q = qkv[:, :_D]
        k = qkv[:, _D:2 * _D]
        v = qkv[:, 2 * _D:]
        nrm = jnp.sqrt(jnp.sum(k * k, axis=1, keepdims=True))
        kn = (k / jnp.maximum(nrm, 1e-12)).astype(jnp.bfloat16)
        sim = lax.dot_general(kn, kn, (((1,), (1,)), ((), ())),
                              preferred_element_type=jnp.float32)
        sim = jnp.where(diag, sim - 1e9, sim)
        work = sim
        sel = jnp.zeros((_N, _N), jnp.bool_)
        for _ in range(_KNN):
            m = jnp.max(work, axis=1, keepdims=True)
            cand = jnp.where(work == m, col, _N)
            fi = jnp.min(cand, axis=1, keepdims=True)
            oh = col == fi
            sel = jnp.logical_or(sel, oh)
            work = jnp.where(oh, _NEG, work)
        scores = lax.dot_general(q, k, (((1,), (1,)), ((), ())),
                                 preferred_element_type=jnp.float32,
                                 precision=_HI) * inv_sqrt_d
        ms = jnp.where(sel, scores, _NEG)
        mx = jnp.max(ms, axis=1, keepdims=True)
        e = jnp.exp(ms - mx)
        p = e / jnp.sum(e, axis=1, keepdims=True)
        attn_out = jnp.dot(p, v, preferred_element_type=jnp.float32,
                           precision=_HI)
        h = h + _bdot(attn_out, wout_ref[l]) + bout_ref[l]
    out_ref[0] = h
    out_rn_ref[0] = h.astype(jnp.bfloat16).astype(jnp.float32)


def _decoder_body(emb_ref, emb_rn_ref, coords_ref, g_ref, bn_ref, w1_ref,
                  b1_ref, w2_ref, b2_ref, wq_ref, bq_ref, out_ref):
    n_iota = lax.broadcasted_iota(jnp.int32, (_B, _N), 1)

    visited0 = (n_iota == 0).astype(jnp.float32)
    token0 = emb_ref[:, 0, :]
    cur0 = coords_ref[:, 0, :]
    tour0 = jnp.zeros((_B, 1), jnp.float32)

    def step(_, carry):
        visited, token, tour, cur = carry
        h = token
        for l in range(_DEC_L):
            mu = jnp.mean(h, axis=1, keepdims=True)
            xc = h - mu
            var = jnp.mean(xc * xc, axis=1, keepdims=True)
            hn = xc / jnp.sqrt(var + 1e-5) * g_ref[l] + bn_ref[l]
            z = _bdot(hn, w1_ref[l]) + b1_ref[l]
            z = jax.nn.silu(z)
            y = _bdot(z, w2_ref[l]) + b2_ref[l]
            h = h + y
        qv = _bdot(h, wq_ref[...]) + bq_ref[...]
        qvb = qv.astype(jnp.bfloat16).astype(jnp.float32)

        logits = jnp.sum(emb_rn_ref[...] * qvb[:, None, :], axis=2)

        ml = jnp.where(visited > 0.5, -1e9, logits)
        mx = jnp.max(ml, axis=1, keepdims=True)
        cand = jnp.where(ml == mx, n_iota, _N)
        nxt = jnp.min(cand, axis=1, keepdims=True)
        onehot = (n_iota == nxt).astype(jnp.float32)
        visited = jnp.maximum(visited, onehot)

        token = jnp.sum(emb_ref[...] * onehot[:, :, None], axis=1)
        nxy = jnp.sum(coords_ref[...] * onehot[:, :, None], axis=1)
        diff = cur - nxy
        tour = tour + jnp.sqrt(jnp.sum(diff * diff, axis=1, keepdims=True))
        return (visited, token, tour, nxy)

    carry = lax.fori_loop(0, _N - 1, step, (visited0, token0, tour0, cur0))
    _, _, tour, cur = carry
    diff = cur - coords_ref[:, 0, :]
    tour = tour + jnp.sqrt(jnp.sum(diff * diff, axis=1, keepdims=True))
    out_ref[...] = jnp.broadcast_to(tour, (_B, 128))


def kernel(coords, params, greedy):
    enc = params["enc"]
    bf = jnp.bfloat16
    wqkv = jnp.stack([lp["Wqkv"] for lp in enc]).astype(bf)
    bqkv = jnp.stack([lp["bqkv"] for lp in enc])[:, None, :]
    wout = jnp.stack([lp["Wout"] for lp in enc]).astype(bf)
    bout = jnp.stack([lp["bout"] for lp in enc])[:, None, :]

    node_emb, node_emb_rn = pl.pallas_call(
        _encoder_body,
        grid=(_B,),
        in_specs=[
            pl.BlockSpec((1, _N, 2), lambda i: (i, 0, 0)),
            pl.BlockSpec((2, _D), lambda i: (0, 0)),
            pl.BlockSpec((1, _D), lambda i: (0, 0)),
            pl.BlockSpec((_ENC_L, _D, 3 * _D), lambda i: (0, 0, 0)),
            pl.BlockSpec((_ENC_L, 1, 3 * _D), lambda i: (0, 0, 0)),
            pl.BlockSpec((_ENC_L, _D, _D), lambda i: (0, 0, 0)),
            pl.BlockSpec((_ENC_L, 1, _D), lambda i: (0, 0, 0)),
        ],
        out_specs=[
            pl.BlockSpec((1, _N, _D), lambda i: (i, 0, 0)),
            pl.BlockSpec((1, _N, _D), lambda i: (i, 0, 0)),
        ],
        out_shape=[
            jax.ShapeDtypeStruct((_B, _N, _D), jnp.float32),
            jax.ShapeDtypeStruct((_B, _N, _D), jnp.float32),
        ],
    )(coords, params["W_in"].astype(bf), params["b_in"][None, :],
      wqkv, bqkv, wout, bout)

    dec = params["dec"]
    g = jnp.stack([p["g"] for p in dec])[:, None, :]
    bn = jnp.stack([p["b"] for p in dec])[:, None, :]
    w1 = jnp.stack([p["W1"] for p in dec]).astype(bf)
    b1 = jnp.stack([p["b1"] for p in dec])[:, None, :]
    w2 = jnp.stack([p["W2"] for p in dec]).astype(bf)
    b2 = jnp.stack([p["b2"] for p in dec])[:, None, :]

    tour_pad = pl.pallas_call(
        _decoder_body,
        out_shape=jax.ShapeDtypeStruct((_B, 128), jnp.float32),
    )(node_emb, node_emb_rn, coords, g, bn, w1, b1, w2, b2,
      params["Wq"].astype(bf), params["bq"][None, :])

    tour = tour_pad[:, 0]
    zeros = jnp.zeros((_B,), jnp.float32)
    return zeros, zeros, tour
